# baseline (speedup 1.0000x reference)
import jax
import jax.numpy as jnp
from jax.experimental import pallas as pl
from jax.experimental.pallas import tpu as pltpu

N = 4096
F = 256
BM = 512
M = N // BM
STEPS = 3 * M
DMA_STEPS = 2 * M
NB = 2
CC = 8
KC = N // CC


def _mm(a, b):
    return jax.lax.dot_general(
        a, b, (((1,), (0,)), ((), ())),
        precision=jax.lax.Precision.DEFAULT,
        preferred_element_type=jnp.float32,
    )


def _block_copy(adj_hbm, buf_ref, sem, g, slot, j):
    layer = jnp.where(g >= M, 1, 0)
    row = jax.lax.rem(g, M) * BM
    return pltpu.make_async_copy(
        adj_hbm.at[layer, pl.ds(row, BM), pl.ds(j * KC, KC)],
        buf_ref.at[slot, :, pl.ds(j * KC, KC)],
        sem.at[slot, j],
    )


def _start_block(adj_hbm, buf_ref, sem, g, slot):
    for j in range(CC):
        _block_copy(adj_hbm, buf_ref, sem, g, slot, j).start()


def _wait_block(adj_hbm, buf_ref, sem, g, slot):
    for j in range(CC):
        _block_copy(adj_hbm, buf_ref, sem, g, slot, j).wait()


def _fused_kernel(adj_hbm, x_hbm, w1_ref, b1_ref, w2_ref, b2_ref,
                  wsvm_ref, bsvm_ref, out_ref,
                  buf_ref, y_ref, h1_ref, cache_ref,
                  acc3_ref, sem, xsem):
    y3_ref = h1_ref
    g = pl.program_id(0)
    slot = jax.lax.rem(g, NB)
    row = jax.lax.rem(g, M) * BM

    @pl.when(g == 0)
    def _():
        cp_x = pltpu.make_async_copy(x_hbm, acc3_ref, xsem)
        cp_x.start()
        for r in range(NB):
            _start_block(adj_hbm, buf_ref, sem, r, r)
        cp_x.wait()
        y_ref[...] = _mm(acc3_ref[...], w1_ref[...])

    @pl.when(g == M)
    def _():
        y_ref[...] = _mm(h1_ref[...], w2_ref[...].astype(jnp.bfloat16))
        acc3_ref[...] = jnp.zeros_like(acc3_ref)

    @pl.when(g < DMA_STEPS)
    def _():
        _wait_block(adj_hbm, buf_ref, sem, g, slot)
        acc = _mm(buf_ref[slot], y_ref[...])

        @pl.when(g < M)
        def _():
            h1_ref[pl.ds(row, BM), :] = jnp.maximum(
                acc + b1_ref[...], 0.0).astype(jnp.bfloat16)

        @pl.when(g >= M)
        def _():
            t = g - M
            h2_blk = jnp.maximum(acc + b2_ref[...], 0.0).astype(jnp.bfloat16)
            cache_ref[t] = buf_ref[slot].astype(jnp.bfloat16)
            y3_ref[pl.ds(row, BM), :] = _mm(
                h2_blk, w2_ref[...].astype(jnp.bfloat16)
            ).astype(jnp.bfloat16)
            for m in range(M):
                @pl.when(m <= t)
                def _(m=m):
                    acc3_ref[pl.ds(m * BM, BM), :] += _mm(
                        cache_ref[m, :, pl.ds(row, BM)],
                        y3_ref[pl.ds(row, BM), :],
                    )

            for j in range(M):
                @pl.when(j < t)
                def _(j=j):
                    acc3_ref[pl.ds(row, BM), :] += _mm(
                        cache_ref[t, :, pl.ds(j * BM, BM)],
                        y3_ref[pl.ds(j * BM, BM), :],
                    )

        @pl.when(g + NB < DMA_STEPS)
        def _():
            _start_block(adj_hbm, buf_ref, sem, g + NB, slot)

    @pl.when(g >= DMA_STEPS)
    def _():
        h = jnp.maximum(acc3_ref[pl.ds(row, BM), :] + b2_ref[...], 0.0)
        logits = _mm(h, wsvm_ref[...]) + bsvm_ref[...]
        mx = jnp.max(logits, axis=1, keepdims=True)
        shifted = logits - mx
        lse = jnp.log(jnp.sum(jnp.exp(shifted), axis=1, keepdims=True))
        out_ref[...] = shifted - lse


@jax.jit
def kernel(x, adj, W1, b1, W2, b2, Wsvm, bsvm):
    nclass = Wsvm.shape[1]
    return pl.pallas_call(
        _fused_kernel,
        grid=(STEPS,),
        in_specs=[
            pl.BlockSpec(memory_space=pltpu.MemorySpace.HBM),
            pl.BlockSpec(memory_space=pltpu.MemorySpace.HBM),
            pl.BlockSpec((F, F), lambda g: (0, 0)),
            pl.BlockSpec((1, F), lambda g: (0, 0)),
            pl.BlockSpec((F, F), lambda g: (0, 0)),
            pl.BlockSpec((1, F), lambda g: (0, 0)),
            pl.BlockSpec((F, nclass), lambda g: (0, 0)),
            pl.BlockSpec((1, nclass), lambda g: (0, 0)),
        ],
        out_specs=pl.BlockSpec((BM, nclass), lambda g: (g % M, 0)),
        out_shape=jax.ShapeDtypeStruct((N, nclass), jnp.float32),
        compiler_params=pltpu.CompilerParams(
            vmem_limit_bytes=64 * 1024 * 1024,
        ),
        scratch_shapes=[
            pltpu.VMEM((NB, BM, N), jnp.float32),
            pltpu.VMEM((N, F), jnp.float32),
            pltpu.VMEM((N, F), jnp.bfloat16),
            pltpu.VMEM((M, BM, N), jnp.bfloat16),
            pltpu.VMEM((N, F), jnp.float32),
            pltpu.SemaphoreType.DMA((NB, CC)),
            pltpu.SemaphoreType.DMA,
        ],
    )(adj, x, W1, b1.reshape(1, F), W2, b2.reshape(1, F),
      Wsvm, bsvm.reshape(1, nclass))


# speedup vs baseline: 1.1429x; 1.1429x over previous
import jax
import jax.numpy as jnp
from jax.experimental import pallas as pl
from jax.experimental.pallas import tpu as pltpu

N = 4096
F = 256
BM = 512
M = N // BM
STEPS = 3 * M
DMA_STEPS = 2 * M
NB = 2
CC = 8
KC = N // CC


def _mm(a, b):
    return jax.lax.dot_general(
        a, b, (((1,), (0,)), ((), ())),
        precision=jax.lax.Precision.DEFAULT,
        preferred_element_type=jnp.float32,
    )


def _block_copy(adj_hbm, buf_ref, sem, g, slot, j):
    layer = jnp.where(g >= M, 1, 0)
    row = jax.lax.rem(g, M) * BM
    return pltpu.make_async_copy(
        adj_hbm.at[layer, pl.ds(row, BM), pl.ds(j * KC, KC)],
        buf_ref.at[slot, :, pl.ds(j * KC, KC)],
        sem.at[slot, j],
    )


def _start_block(adj_hbm, buf_ref, sem, g, slot):
    for j in range(CC):
        _block_copy(adj_hbm, buf_ref, sem, g, slot, j).start()


def _wait_block(adj_hbm, buf_ref, sem, g, slot):
    for j in range(CC):
        _block_copy(adj_hbm, buf_ref, sem, g, slot, j).wait()


def _fused_kernel(adj_hbm, x_hbm, w1_ref, b1_ref, w2_ref, b2_ref,
                  wsvm_ref, bsvm_ref, out_ref,
                  buf_ref, y_ref, y3_ref, h1_ref, h2_ref, cache_ref,
                  sem, xsem):
    g = pl.program_id(0)
    slot = jax.lax.rem(g, NB)
    row = jax.lax.rem(g, M) * BM

    @pl.when(g == 0)
    def _():
        cp_x = pltpu.make_async_copy(x_hbm, h2_ref, xsem)
        cp_x.start()
        for r in range(NB):
            _start_block(adj_hbm, buf_ref, sem, r, r)
        cp_x.wait()
        y_ref[...] = _mm(h2_ref[...], w1_ref[...])

    @pl.when(g == M)
    def _():
        y_ref[...] = jax.lax.dot_general(
            h1_ref[...], w2_ref[...].astype(jnp.bfloat16),
            (((1,), (0,)), ((), ())),
            precision=jax.lax.Precision.DEFAULT,
            preferred_element_type=jnp.float32,
        )

    @pl.when(g == 2 * M)
    def _():
        y3_ref[...] = _mm(h2_ref[...], w2_ref[...]).astype(jnp.bfloat16)

    @pl.when(g < DMA_STEPS)
    def _():
        _wait_block(adj_hbm, buf_ref, sem, g, slot)
        acc = _mm(buf_ref[slot], y_ref[...])

        @pl.when(g < M)
        def _():
            h1_ref[pl.ds(row, BM), :] = jnp.maximum(
                acc + b1_ref[...], 0.0).astype(jnp.bfloat16)

        @pl.when(g >= M)
        def _():
            h2_ref[pl.ds(row, BM), :] = jnp.maximum(acc + b2_ref[...], 0.0)
            cache_ref[jax.lax.rem(g, M)] = buf_ref[slot].astype(jnp.bfloat16)

        @pl.when(g + NB < DMA_STEPS)
        def _():
            _start_block(adj_hbm, buf_ref, sem, g + NB, slot)

    @pl.when(g >= DMA_STEPS)
    def _():
        acc = _mm(cache_ref[jax.lax.rem(g, M)], y3_ref[...])
        h = jnp.maximum(acc + b2_ref[...], 0.0)
        logits = _mm(h, wsvm_ref[...]) + bsvm_ref[...]
        mx = jnp.max(logits, axis=1, keepdims=True)
        shifted = logits - mx
        lse = jnp.log(jnp.sum(jnp.exp(shifted), axis=1, keepdims=True))
        out_ref[...] = shifted - lse


@jax.jit
def kernel(x, adj, W1, b1, W2, b2, Wsvm, bsvm):
    nclass = Wsvm.shape[1]
    return pl.pallas_call(
        _fused_kernel,
        grid=(STEPS,),
        in_specs=[
            pl.BlockSpec(memory_space=pltpu.MemorySpace.HBM),
            pl.BlockSpec(memory_space=pltpu.MemorySpace.HBM),
            pl.BlockSpec((F, F), lambda g: (0, 0)),
            pl.BlockSpec((1, F), lambda g: (0, 0)),
            pl.BlockSpec((F, F), lambda g: (0, 0)),
            pl.BlockSpec((1, F), lambda g: (0, 0)),
            pl.BlockSpec((F, nclass), lambda g: (0, 0)),
            pl.BlockSpec((1, nclass), lambda g: (0, 0)),
        ],
        out_specs=pl.BlockSpec((BM, nclass), lambda g: (g % M, 0)),
        out_shape=jax.ShapeDtypeStruct((N, nclass), jnp.float32),
        compiler_params=pltpu.CompilerParams(
            vmem_limit_bytes=64 * 1024 * 1024,
        ),
        scratch_shapes=[
            pltpu.VMEM((NB, BM, N), jnp.float32),
            pltpu.VMEM((N, F), jnp.float32),
            pltpu.VMEM((N, F), jnp.bfloat16),
            pltpu.VMEM((N, F), jnp.bfloat16),
            pltpu.VMEM((N, F), jnp.float32),
            pltpu.VMEM((M, BM, N), jnp.bfloat16),
            pltpu.SemaphoreType.DMA((NB, CC)),
            pltpu.SemaphoreType.DMA,
        ],
    )(adj, x, W1, b1.reshape(1, F), W2, b2.reshape(1, F),
      Wsvm, bsvm.reshape(1, nclass))
